# Initial kernel scaffold; baseline (speedup 1.0000x reference)
import jax
import jax.numpy as jnp
from jax.experimental import pallas as pl


def kernel(q_sub, q_rel, hidden, path_state, edges, nodes, old_nodes_new_idx, batchsize, rela_embed, Ws_attn, Wr_attn, Wqr_attn_w, Wqr_attn_b, w_alpha_w, w_alpha_b, W_h, W_path_prev, W_path_rel, curvature):
    raise NotImplementedError("write your pallas kernel here")


# speedup vs baseline: 2.3916x; 2.3916x over previous
import functools

import jax
import jax.numpy as jnp
from jax import lax
from jax.experimental import pallas as pl
from jax.experimental.pallas import tpu as pltpu
from jax.experimental.pallas import tpu_sc as plsc

MIN_NORM = 1e-15
BALL_EPS = 0.004
MIN_CURVATURE = 1e-06

N_NODE = 10000
N_EDGE = 320000
D = 128
DP = 64
NPAD = 10112
NBLK = 79
NCORE = 2
NSUB = 16
NW = NCORE * NSUB
EPW = N_EDGE // NW
CH = 80
NCH = EPW // CH
RPT = N_NODE // NSUB
EB = 512
NEB = N_EDGE // EB


def _tanh_c(x):
    return jnp.tanh(jnp.clip(x, -15.0, 15.0))


def _artanh(x):
    x = jnp.clip(x, -1.0 + 1e-05, 1.0 - 1e-05)
    return 0.5 * (jnp.log1p(x) - jnp.log1p(-x))


def _expmap0(u, c):
    sqrt_c = jnp.sqrt(c)
    u_norm = jnp.maximum(jnp.sqrt(jnp.sum(u * u, axis=-1, keepdims=True)), MIN_NORM)
    gamma = _tanh_c(sqrt_c * u_norm) * u / (sqrt_c * u_norm)
    norm = jnp.maximum(jnp.sqrt(jnp.sum(gamma * gamma, axis=-1, keepdims=True)), MIN_NORM)
    maxnorm = (1.0 - BALL_EPS) / sqrt_c
    return jnp.where(norm > maxnorm, gamma / norm * maxnorm, gamma)


def _pre_body(c_ref, hid, rela, path, Ws, Wr, Wq, bq, Wpp, Wpr,
              AS, AR, AQ, PS, PR, HS, HR):
    c = jnp.maximum(c_ref[0, 0], MIN_CURVATURE)
    h = hid[...]
    r = rela[...]
    AS[...] = jax.lax.dot_general(h, Ws[...], (((1,), (0,)), ((), ())),
                                  preferred_element_type=jnp.float32)
    AR[...] = jax.lax.dot_general(r, Wr[...], (((1,), (0,)), ((), ())),
                                  preferred_element_type=jnp.float32)
    AQ[...] = jax.lax.dot_general(r, Wq[...], (((1,), (0,)), ((), ())),
                                  preferred_element_type=jnp.float32) + bq[...]
    PS[...] = jax.lax.dot_general(path[...], Wpp[...], (((1,), (0,)), ((), ())),
                                  preferred_element_type=jnp.float32)
    PR[...] = jax.lax.dot_general(r, Wpr[...], (((1,), (0,)), ((), ())),
                                  preferred_element_type=jnp.float32)
    HS[...] = _expmap0(h, c)
    HR[...] = _expmap0(r, c)


def _precompute(cc, hid_p, rela_p, path_p, Ws, Wr, Wq, bq, Wpp, Wpr):
    f32 = jnp.float32
    full = lambda s: pl.BlockSpec(s, lambda i: (0,) * len(s))
    row128 = pl.BlockSpec((D, D), lambda i: (i, 0))
    row64 = pl.BlockSpec((D, DP), lambda i: (i, 0))
    return pl.pallas_call(
        _pre_body,
        grid=(NBLK,),
        in_specs=[
            pl.BlockSpec(memory_space=pltpu.SMEM),
            row128, row128, row64,
            full((D, D)), full((D, D)), full((D, D)), full((1, D)),
            full((DP, DP)), full((D, DP)),
        ],
        out_specs=[row128, row128, row128, row64, row64, row128, row128],
        out_shape=[
            jax.ShapeDtypeStruct((NPAD, D), f32),
            jax.ShapeDtypeStruct((NPAD, D), f32),
            jax.ShapeDtypeStruct((NPAD, D), f32),
            jax.ShapeDtypeStruct((NPAD, DP), f32),
            jax.ShapeDtypeStruct((NPAD, DP), f32),
            jax.ShapeDtypeStruct((NPAD, D), f32),
            jax.ShapeDtypeStruct((NPAD, D), f32),
        ],
    )(cc, hid_p, rela_p, path_p, Ws, Wr, Wq, bq, Wpp, Wpr)


def _gather_body(qrel_hbm, ridx_hbm, sub_hbm, rel_hbm,
                 AS_hbm, AR_hbm, AQ_hbm, HS_hbm, HR_hbm, PS_hbm, PR_hbm,
                 ASg, ARg, AQg, HSg, HRg, PSg, PRg,
                 ridx_v, sub_v, rel_v, qi_v,
                 b_as, b_ar, b_aq, b_hs, b_hr, b_ps, b_pr, sem):
    cid = lax.axis_index("c")
    sid = lax.axis_index("s")
    wid = sid * NCORE + cid
    base = wid * EPW
    pltpu.sync_copy(ridx_hbm.at[pl.ds(base, EPW)], ridx_v)
    pltpu.sync_copy(sub_hbm.at[pl.ds(base, EPW)], sub_v)
    pltpu.sync_copy(rel_hbm.at[pl.ds(base, EPW)], rel_v)
    pltpu.async_copy(qrel_hbm.at[ridx_v], qi_v, sem).wait()

    def step(k, _):
        off = k * CH
        cs = []
        cs.append(pltpu.async_copy(AS_hbm.at[sub_v.at[pl.ds(off, CH)]], b_as, sem))
        cs.append(pltpu.async_copy(AR_hbm.at[rel_v.at[pl.ds(off, CH)]], b_ar, sem))
        cs.append(pltpu.async_copy(AQ_hbm.at[qi_v.at[pl.ds(off, CH)]], b_aq, sem))
        cs.append(pltpu.async_copy(HS_hbm.at[sub_v.at[pl.ds(off, CH)]], b_hs, sem))
        cs.append(pltpu.async_copy(HR_hbm.at[rel_v.at[pl.ds(off, CH)]], b_hr, sem))
        cs.append(pltpu.async_copy(PS_hbm.at[sub_v.at[pl.ds(off, CH)]], b_ps, sem))
        cs.append(pltpu.async_copy(PR_hbm.at[rel_v.at[pl.ds(off, CH)]], b_pr, sem))
        for h in cs:
            h.wait()
        pltpu.sync_copy(b_as, ASg.at[pl.ds(base + off, CH)])
        pltpu.sync_copy(b_ar, ARg.at[pl.ds(base + off, CH)])
        pltpu.sync_copy(b_aq, AQg.at[pl.ds(base + off, CH)])
        pltpu.sync_copy(b_hs, HSg.at[pl.ds(base + off, CH)])
        pltpu.sync_copy(b_hr, HRg.at[pl.ds(base + off, CH)])
        pltpu.sync_copy(b_ps, PSg.at[pl.ds(base + off, CH)])
        pltpu.sync_copy(b_pr, PRg.at[pl.ds(base + off, CH)])
        return 0

    lax.fori_loop(0, NCH, step, 0)


def _edge_gather(qrel, ridx, sub, rel, AS, AR, AQ, HS, HR, PS, PR):
    f32 = jnp.float32
    i32 = jnp.int32
    mesh = plsc.VectorSubcoreMesh(core_axis_name="c", subcore_axis_name="s",
                                  num_cores=NCORE, num_subcores=NSUB)
    out_type = (
        jax.ShapeDtypeStruct((N_EDGE, D), f32),
        jax.ShapeDtypeStruct((N_EDGE, D), f32),
        jax.ShapeDtypeStruct((N_EDGE, D), f32),
        jax.ShapeDtypeStruct((N_EDGE, D), f32),
        jax.ShapeDtypeStruct((N_EDGE, D), f32),
        jax.ShapeDtypeStruct((N_EDGE, DP), f32),
        jax.ShapeDtypeStruct((N_EDGE, DP), f32),
    )
    scratch = [
        pltpu.VMEM((EPW,), i32),
        pltpu.VMEM((EPW,), i32),
        pltpu.VMEM((EPW,), i32),
        pltpu.VMEM((EPW,), i32),
        pltpu.VMEM((CH, D), f32),
        pltpu.VMEM((CH, D), f32),
        pltpu.VMEM((CH, D), f32),
        pltpu.VMEM((CH, D), f32),
        pltpu.VMEM((CH, D), f32),
        pltpu.VMEM((CH, DP), f32),
        pltpu.VMEM((CH, DP), f32),
        pltpu.SemaphoreType.DMA,
    ]
    k = pl.kernel(_gather_body, out_type=out_type, mesh=mesh, scratch_types=scratch,
                  compiler_params=pltpu.CompilerParams(use_tc_tiling_on_sc=False))
    return k(qrel, ridx, sub, rel, AS, AR, AQ, HS, HR, PS, PR)


def _edge_body(c_ref, wa, wb_ref, asg, arg, aqg, hsg, hrg, psg, prg, msg, pe):
    c = jnp.maximum(c_ref[0, 0], MIN_CURVATURE)
    sqrt_c = jnp.sqrt(c)
    att = jnp.maximum(asg[...] + arg[...] + aqg[...], 0.0)
    logit = jax.lax.dot_general(att, wa[...], (((1,), (0,)), ((), ())),
                                preferred_element_type=jnp.float32) + wb_ref[0, 0]
    alpha = jax.nn.sigmoid(logit)
    hs = hsg[...]
    hr = hrg[...]
    x2 = jnp.sum(hs * hs, axis=-1, keepdims=True)
    y2 = jnp.sum(hr * hr, axis=-1, keepdims=True)
    xy = jnp.sum(hs * hr, axis=-1, keepdims=True)
    A = 1.0 + 2.0 * c * xy + c * y2
    B = 1.0 - c * x2
    den = jnp.maximum(1.0 + 2.0 * c * xy + c * c * x2 * y2, MIN_NORM)
    r2 = A * A * x2 + 2.0 * A * B * xy + B * B * y2
    nm0 = jnp.maximum(jnp.sqrt(jnp.maximum(r2, 0.0)) / den, MIN_NORM)
    maxnorm = (1.0 - BALL_EPS) / sqrt_c
    s1 = jnp.where(nm0 > maxnorm, maxnorm / nm0, 1.0) / den
    n2 = jnp.maximum(jnp.minimum(nm0, maxnorm), MIN_NORM)
    factor = _artanh(sqrt_c * n2) / (n2 * sqrt_c)
    p = alpha * factor * s1 * A
    q = alpha * factor * s1 * B
    msg[...] = p * hs + q * hr
    pe[...] = alpha * _tanh_c(psg[...] + prg[...])


def _edge_math(cc, wa, wb, ASg, ARg, AQg, HSg, HRg, PSg, PRg):
    f32 = jnp.float32
    blk128 = pl.BlockSpec((EB, D), lambda i: (i, 0))
    blk64 = pl.BlockSpec((EB, DP), lambda i: (i, 0))
    return pl.pallas_call(
        _edge_body,
        grid=(NEB,),
        in_specs=[
            pl.BlockSpec(memory_space=pltpu.SMEM),
            pl.BlockSpec((D, 1), lambda i: (0, 0)),
            pl.BlockSpec(memory_space=pltpu.SMEM),
            blk128, blk128, blk128, blk128, blk128, blk64, blk64,
        ],
        out_specs=[blk128, blk64],
        out_shape=[jax.ShapeDtypeStruct((N_EDGE, D), f32),
                   jax.ShapeDtypeStruct((N_EDGE, DP), f32)],
    )(cc, wa, wb, ASg, ARg, AQg, HSg, HRg, PSg, PRg)


def _scatter_body(msg_hbm, obj_hbm, z_hbm, part,
                  obj_v, buf, acc):
    cid = lax.axis_index("c")
    sid = lax.axis_index("s")
    wslot = cid * NSUB + sid
    ebase = wslot * EPW
    pltpu.sync_copy(z_hbm.at[pl.ds(sid * RPT, RPT)], acc.at[pl.ds(sid * RPT, RPT)])
    pltpu.sync_copy(obj_hbm.at[wslot], obj_v)
    plsc.subcore_barrier()

    def step(k, _):
        off = k * CH
        pltpu.sync_copy(msg_hbm.at[pl.ds(ebase + off, CH)], buf)
        pltpu.sync_copy(buf, acc.at[obj_v.at[k]], add=True)
        return 0

    lax.fori_loop(0, NCH, step, 0)
    plsc.subcore_barrier()
    pltpu.sync_copy(acc.at[pl.ds(sid * RPT, RPT)],
                    part.at[cid].at[pl.ds(sid * RPT, RPT)])


def _scatter(msg, obj3, z, width):
    f32 = jnp.float32
    i32 = jnp.int32
    mesh = plsc.VectorSubcoreMesh(core_axis_name="c", subcore_axis_name="s",
                                  num_cores=NCORE, num_subcores=NSUB)
    out_type = jax.ShapeDtypeStruct((NCORE, N_NODE, width), f32)
    scratch = [
        pltpu.VMEM((NCH, CH), i32),
        pltpu.VMEM((CH, width), f32),
        pltpu.VMEM_SHARED((N_NODE, width), f32),
    ]
    k = pl.kernel(_scatter_body, out_type=out_type, mesh=mesh, scratch_types=scratch,
                  compiler_params=pltpu.CompilerParams(use_tc_tiling_on_sc=False))
    return k(msg, obj3, z)


def _final_body(c_ref, Wh, mp, pp, out1, out2):
    c = jnp.maximum(c_ref[0, 0], MIN_CURVATURE)
    sqrt_c = jnp.sqrt(c)
    magg = mp[0] + mp[1]
    a = jax.lax.dot_general(magg, Wh[...], (((1,), (0,)), ((), ())),
                            preferred_element_type=jnp.float32)
    h = _expmap0(a, c)
    n = jnp.maximum(jnp.sqrt(jnp.sum(h * h, axis=-1, keepdims=True)), MIN_NORM)
    out1[...] = h / n / sqrt_c * _artanh(sqrt_c * n)
    out2[...] = pp[0] + pp[1]


def _final(cc, Wh, mpart, ppart):
    f32 = jnp.float32
    FB = 80
    return pl.pallas_call(
        _final_body,
        grid=(N_NODE // FB,),
        in_specs=[
            pl.BlockSpec(memory_space=pltpu.SMEM),
            pl.BlockSpec((D, D), lambda i: (0, 0)),
            pl.BlockSpec((NCORE, FB, D), lambda i: (0, i, 0)),
            pl.BlockSpec((NCORE, FB, DP), lambda i: (0, i, 0)),
        ],
        out_specs=[pl.BlockSpec((FB, D), lambda i: (i, 0)),
                   pl.BlockSpec((FB, DP), lambda i: (i, 0))],
        out_shape=[jax.ShapeDtypeStruct((N_NODE, D), f32),
                   jax.ShapeDtypeStruct((N_NODE, DP), f32)],
    )(cc, Wh, mpart, ppart)


def kernel(q_sub, q_rel, hidden, path_state, edges, nodes, old_nodes_new_idx,
           batchsize, rela_embed, Ws_attn, Wr_attn, Wqr_attn_w, Wqr_attn_b,
           w_alpha_w, w_alpha_b, W_h, W_path_prev, W_path_rel, curvature):
    f32 = jnp.float32
    i32 = jnp.int32
    cc = jnp.reshape(jnp.asarray(curvature, f32), (1, 1))
    wb = jnp.reshape(jnp.asarray(w_alpha_b, f32), (1, 1))

    hid_p = jnp.zeros((NPAD, D), f32).at[:N_NODE].set(hidden)
    rela_p = jnp.zeros((NPAD, D), f32).at[:rela_embed.shape[0]].set(rela_embed)
    path_p = jnp.zeros((NPAD, DP), f32).at[:N_NODE].set(path_state)
    sub = jnp.asarray(edges[:, 4], i32)
    rel = jnp.asarray(edges[:, 2], i32)
    obj = jnp.asarray(edges[:, 5], i32)
    ridx = jnp.asarray(edges[:, 0], i32)
    qrel = jnp.asarray(q_rel, i32)
    obj3 = jnp.reshape(obj, (NW, NCH, CH))
    z128 = jnp.zeros((N_NODE, D), f32)
    z64 = jnp.zeros((N_NODE, DP), f32)

    AS, AR, AQ, PS, PR, HS, HR = _precompute(
        cc, hid_p, rela_p, path_p, Ws_attn, Wr_attn, Wqr_attn_w,
        jnp.reshape(Wqr_attn_b, (1, D)), W_path_prev, W_path_rel)

    ASg, ARg, AQg, HSg, HRg, PSg, PRg = _edge_gather(
        qrel, ridx, sub, rel, AS, AR, AQ, HS, HR, PS, PR)

    msg, pe = _edge_math(cc, jnp.reshape(w_alpha_w, (D, 1)), wb,
                         ASg, ARg, AQg, HSg, HRg, PSg, PRg)

    mpart = _scatter(msg, obj3, z128, D)
    ppart = _scatter(pe, obj3, z64, DP)

    out1, out2 = _final(cc, W_h, mpart, ppart)
    return (out1, out2)
